# baseline (speedup 1.0000x reference)
import functools

import jax
import jax.numpy as jnp
from jax import lax
from jax.experimental import pallas as pl
from jax.experimental.pallas import tpu as pltpu
from jax.experimental.pallas import tpu_sc as plsc

N = 10000
E = 320000
DN = 128
DE = 16
H = 80

NC, NS = 2, 16
NW = NC * NS
CH = 80
NCH_A = 77
NCH_B = 48
E_A = NW * NCH_A * CH
E_B = NW * NCH_B * CH
ROWS_A = E_A // CH
HP = 128
F32 = jnp.float32
BF = jnp.bfloat16

_SC_PARAMS = pltpu.CompilerParams(use_tc_tiling_on_sc=False)


@functools.cache
def _sc_mesh():
    return plsc.VectorSubcoreMesh(
        core_axis_name="c", subcore_axis_name="s", num_cores=NC, num_subcores=NS)


@functools.cache
def _gather_sum_kernel(nchunk, row0):
    per_w = nchunk * CH

    def body(a_hbm, b_hbm, dst_hbm, src_hbm, out_hbm,
             idxd, idxs, a0, b0, a1, b1, sem0, sem1):
        wid = lax.axis_index("s") * NC + lax.axis_index("c")
        r0 = row0 + wid * nchunk
        pltpu.sync_copy(dst_hbm.at[pl.ds(r0, nchunk)], idxd)
        pltpu.sync_copy(src_hbm.at[pl.ds(r0, nchunk)], idxs)
        base = wid * per_w

        def fire(j, ab, bb, sem):
            pltpu.async_copy(a_hbm.at[idxd.at[j]], ab, sem)
            pltpu.async_copy(b_hbm.at[idxs.at[j]], bb, sem)

        def drain(ab, bb, sem):
            pltpu.make_async_copy(a_hbm.at[idxd.at[0]], ab, sem).wait()
            pltpu.make_async_copy(b_hbm.at[idxs.at[0]], bb, sem).wait()

        def addstore(ab, bb, j):
            def row(i, c2):
                for q in range(HP // 32):
                    sl = pl.ds(q * 32, 32)
                    ab[i, sl] = ab[i, sl] + bb[i, sl]
                return c2

            lax.fori_loop(0, CH, row, 0)
            pltpu.sync_copy(ab, out_hbm.at[pl.ds(base + j * CH, CH)])

        fire(0, a0, b0, sem0)
        fire(1, a1, b1, sem1)

        def pair(i, carry):
            j0 = 2 * i
            drain(a0, b0, sem0)
            addstore(a0, b0, j0)

            @pl.when(j0 + 2 < nchunk)
            def _():
                fire(j0 + 2, a0, b0, sem0)

            drain(a1, b1, sem1)
            addstore(a1, b1, j0 + 1)

            @pl.when(j0 + 3 < nchunk)
            def _():
                fire(j0 + 3, a1, b1, sem1)

            return carry

        lax.fori_loop(0, nchunk // 2, pair, 0)
        if nchunk % 2:
            drain(a0, b0, sem0)
            addstore(a0, b0, nchunk - 1)

    return pl.kernel(
        body,
        out_type=jax.ShapeDtypeStruct((NW * per_w, HP), BF),
        mesh=_sc_mesh(),
        compiler_params=_SC_PARAMS,
        scratch_types=[
            pltpu.VMEM((nchunk, CH), jnp.int32),
            pltpu.VMEM((nchunk, CH), jnp.int32),
            pltpu.VMEM((CH, HP), BF),
            pltpu.VMEM((CH, HP), BF),
            pltpu.VMEM((CH, HP), BF),
            pltpu.VMEM((CH, HP), BF),
            pltpu.SemaphoreType.DMA,
            pltpu.SemaphoreType.DMA,
        ],
    )


@functools.cache
def _scatter_add_kernel(ncha, nchb):
    def body(ea_hbm, eb_hbm, dst_hbm, zero_hbm, out_hbm,
             idxa, idxb, e0, e1, shared, sem0, sem1):
        c = lax.axis_index("c")
        s = lax.axis_index("s")
        wid = s * NC + c

        @pl.when(s == 0)
        def _():
            pltpu.sync_copy(zero_hbm, shared)

        plsc.subcore_barrier()
        pltpu.sync_copy(dst_hbm.at[pl.ds(wid * ncha, ncha)], idxa)
        pltpu.sync_copy(dst_hbm.at[pl.ds(ROWS_A + wid * nchb, nchb)], idxb)

        def phase(e_hbm, idx, nch):
            base = wid * nch * CH

            def fire(j, eb, sem):
                pltpu.async_copy(e_hbm.at[pl.ds(base + j * CH, CH)], eb, sem)

            def drain(eb, sem):
                pltpu.make_async_copy(e_hbm.at[pl.ds(0, CH)], eb, sem).wait()

            def scat(eb, j):
                pltpu.sync_copy(eb, shared.at[idx.at[j]], add=True)

            fire(0, e0, sem0)
            fire(1, e1, sem1)

            def pair(i, carry):
                j0 = 2 * i
                drain(e0, sem0)
                scat(e0, j0)

                @pl.when(j0 + 2 < nch)
                def _():
                    fire(j0 + 2, e0, sem0)

                drain(e1, sem1)
                scat(e1, j0 + 1)

                @pl.when(j0 + 3 < nch)
                def _():
                    fire(j0 + 3, e1, sem1)

                return carry

            lax.fori_loop(0, nch // 2, pair, 0)
            if nch % 2:
                drain(e0, sem0)
                scat(e0, nch - 1)

        phase(ea_hbm, idxa, ncha)
        phase(eb_hbm, idxb, nchb)
        plsc.subcore_barrier()

        @pl.when(s == 0)
        def _():
            pltpu.sync_copy(shared, out_hbm.at[c])

    return pl.kernel(
        body,
        out_type=jax.ShapeDtypeStruct((NC, N, DE), F32),
        mesh=_sc_mesh(),
        compiler_params=_SC_PARAMS,
        scratch_types=[
            pltpu.VMEM((ncha, CH), jnp.int32),
            pltpu.VMEM((nchb, CH), jnp.int32),
            pltpu.VMEM((CH, DE), F32),
            pltpu.VMEM((CH, DE), F32),
            pltpu.VMEM_SHARED((N, DE), F32),
            pltpu.SemaphoreType.DMA,
            pltpu.SemaphoreType.DMA,
        ],
    )


def _dot(a, b):
    return jnp.dot(a, b, preferred_element_type=F32)


def _full(a):
    return pl.BlockSpec(a.shape, lambda i: (0,) * a.ndim)


def _project_body(x_ref, wd_ref, ws_ref, a_ref, b_ref):
    xb = x_ref[...]
    a_ref[...] = _dot(xb, wd_ref[...]).astype(BF)
    b_ref[...] = _dot(xb, ws_ref[...]).astype(BF)


def _node_project(x, wd, ws, blk=2000):
    return pl.pallas_call(
        _project_body,
        grid=(N // blk,),
        in_specs=[pl.BlockSpec((blk, DN), lambda i: (i, 0)),
                  _full(wd), _full(ws)],
        out_specs=[pl.BlockSpec((blk, HP), lambda i: (i, 0)),
                   pl.BlockSpec((blk, HP), lambda i: (i, 0))],
        out_shape=[jax.ShapeDtypeStruct((N, HP), BF),
                   jax.ShapeDtypeStruct((N, HP), BF)],
    )(x, wd, ws)


def _dot_tl(at, b):
    return lax.dot_general(at, b, (((0,), (0,)), ((), ())),
                           preferred_element_type=F32)


def _edge_body(g_ref, ea_ref, we_ref, b1_ref, w2_ref, b2_ref, w3_ref, b3_ref,
               w4_ref, b4_ref, out_ref):
    h = (g_ref[...].astype(F32) + _dot_tl(ea_ref[...].astype(BF), we_ref[...])
         + b1_ref[...])
    h = jnp.maximum(h, 0.0)
    h = jnp.maximum(_dot(h.astype(BF), w2_ref[...]) + b2_ref[...], 0.0)
    h = jnp.maximum(_dot(h.astype(BF), w3_ref[...]) + b3_ref[...], 0.0)
    out_ref[...] = _dot(h.astype(BF), w4_ref[...]) + b4_ref[...]


def _edge_mlp(g, ea_t, weights, blk=2560):
    ne = g.shape[0]
    return pl.pallas_call(
        _edge_body,
        grid=(ne // blk,),
        in_specs=[pl.BlockSpec((blk, HP), lambda i: (i, 0)),
                  pl.BlockSpec((DE, blk), lambda i: (0, i))]
                 + [_full(w) for w in weights],
        out_specs=pl.BlockSpec((blk, DE), lambda i: (i, 0)),
        out_shape=jax.ShapeDtypeStruct((ne, DE), F32),
    )(g, ea_t, *weights)


def _node_fused_body(x_ref, agg_ref, wx_ref, wa_ref, b1_ref, w2_ref, b2_ref,
                     w3_ref, b3_ref, w4_ref, b4_ref, wd_ref, ws_ref,
                     a_ref, b_ref):
    agg = agg_ref[0] + agg_ref[1]
    h = _dot(x_ref[...], wx_ref[...]) + _dot(agg, wa_ref[...]) + b1_ref[...]
    h = jnp.maximum(h, 0.0)
    h = jnp.maximum(_dot(h, w2_ref[...]) + b2_ref[...], 0.0)
    h = jnp.maximum(_dot(h, w3_ref[...]) + b3_ref[...], 0.0)
    x1 = _dot(h, w4_ref[...]) + b4_ref[...]
    a_ref[...] = _dot(x1, wd_ref[...]).astype(BF)
    b_ref[...] = _dot(x1, ws_ref[...]).astype(BF)


def _node_fused(x, agg2, weights, blk=2000):
    return pl.pallas_call(
        _node_fused_body,
        grid=(N // blk,),
        in_specs=[pl.BlockSpec((blk, DN), lambda i: (i, 0)),
                  pl.BlockSpec((NC, blk, DE), lambda i: (0, i, 0))]
                 + [_full(w) for w in weights],
        out_specs=[pl.BlockSpec((blk, HP), lambda i: (i, 0)),
                   pl.BlockSpec((blk, HP), lambda i: (i, 0))],
        out_shape=[jax.ShapeDtypeStruct((N, HP), BF),
                   jax.ShapeDtypeStruct((N, HP), BF)],
    )(x, agg2, *weights)


def _edge_final_body(g_ref, e1_ref, ea_ref, we_ref, b1_ref, w2_ref, b2_ref,
                     w3_ref, b3_ref, w4_ref, b4_ref, va_ref, vb_ref, vc_ref,
                     c1_ref, v2_ref, c2_ref, v3_ref, c3_ref, v4_ref, c4_ref,
                     out_ref):
    e1 = e1_ref[...].astype(BF)
    h = (g_ref[...].astype(F32) + _dot(e1, we_ref[...]) + b1_ref[...])
    h = jnp.maximum(h, 0.0)
    h = jnp.maximum(_dot(h.astype(BF), w2_ref[...]) + b2_ref[...], 0.0)
    h = jnp.maximum(_dot(h.astype(BF), w3_ref[...]) + b3_ref[...], 0.0)
    e2 = _dot(h.astype(BF), w4_ref[...]) + b4_ref[...]
    hw = (_dot_tl(ea_ref[...].astype(BF), va_ref[...]) + _dot(e1, vb_ref[...])
          + _dot(e2.astype(BF), vc_ref[...]) + c1_ref[...])
    hw = jnp.maximum(hw, 0.0)
    hw = jnp.maximum(_dot(hw.astype(BF), v2_ref[...]) + c2_ref[...], 0.0)
    hw = jnp.maximum(_dot(hw.astype(BF), v3_ref[...]) + c3_ref[...], 0.0)
    out_ref[...] = jax.nn.sigmoid(_dot(hw.astype(BF), v4_ref[...]) + c4_ref[...])


def _edge_final(g, e1, ea_t, weights, blk=2560):
    ne = g.shape[0]
    return pl.pallas_call(
        _edge_final_body,
        grid=(ne // blk,),
        in_specs=[pl.BlockSpec((blk, HP), lambda i: (i, 0)),
                  pl.BlockSpec((blk, DE), lambda i: (i, 0)),
                  pl.BlockSpec((DE, blk), lambda i: (0, i))]
                 + [_full(w) for w in weights],
        out_specs=pl.BlockSpec((blk, 1), lambda i: (i, 0)),
        out_shape=jax.ShapeDtypeStruct((ne, 1), F32),
    )(g, e1, ea_t, *weights)


def kernel(x, edge_index, edge_attr, params):
    dst2 = edge_index[1].reshape(E // CH, CH)
    src2 = edge_index[0].reshape(E // CH, CH)
    ea_t = edge_attr.T
    eat_a, eat_b = ea_t[:, :E_A], ea_t[:, E_A:]
    zeros = jnp.zeros((N, DE), F32)

    def padc(w):
        return jnp.concatenate(
            [w, jnp.zeros(w.shape[:-1] + (HP - w.shape[-1],), w.dtype)], -1)

    def padr(w):
        return jnp.concatenate(
            [w, jnp.zeros((HP - w.shape[0],) + w.shape[1:], w.dtype)], 0)

    (r1w1, r1b1), (r1w2, r1b2), (r1w3, r1b3), (r1w4, r1b4) = params['r1']
    (o1w1, o1b1), (o1w2, o1b2), (o1w3, o1b3), (o1w4, o1b4) = params['o1']
    (r2w1, r2b1), (r2w2, r2b2), (r2w3, r2b3), (r2w4, r2b4) = params['r2']
    (ww1, wb1), (ww2, wb2), (ww3, wb3), (ww4, wb4) = params['w']

    a1, b1 = _node_project(x, padc(r1w1[:DN]), padc(r1w1[DN:2 * DN]))
    g1a = _gather_sum_kernel(NCH_A, 0)(a1, b1, dst2, src2)
    g1b = _gather_sum_kernel(NCH_B, ROWS_A)(a1, b1, dst2, src2)
    r1ws = (padc(r1w1[2 * DN:]).astype(BF), padc(r1b1[None]),
            padr(r1w2).astype(BF), r1b2[None], r1w3.astype(BF), r1b3[None],
            r1w4.astype(BF), r1b4[None])
    e1a = _edge_mlp(g1a, eat_a, r1ws)
    e1b = _edge_mlp(g1b, eat_b, r1ws)
    agg2 = _scatter_add_kernel(NCH_A, NCH_B)(e1a, e1b, dst2, zeros)

    a2, b2 = _node_fused(x, agg2,
                         (o1w1[:DN], o1w1[DN:], o1b1[None], o1w2, o1b2[None],
                          o1w3, o1b3[None], o1w4, o1b4[None],
                          padc(r2w1[:DN]), padc(r2w1[DN:2 * DN])))

    g2a = _gather_sum_kernel(NCH_A, 0)(a2, b2, dst2, src2)
    g2b = _gather_sum_kernel(NCH_B, ROWS_A)(a2, b2, dst2, src2)
    weights = (padc(r2w1[2 * DN:]).astype(BF), padc(r2b1[None]),
               padr(r2w2).astype(BF), r2b2[None], r2w3.astype(BF), r2b3[None],
               r2w4.astype(BF), r2b4[None],
               ww1[:DE].astype(BF), ww1[DE:2 * DE].astype(BF),
               ww1[2 * DE:].astype(BF), wb1[None], ww2.astype(BF), wb2[None],
               ww3.astype(BF), wb3[None], ww4.astype(BF), wb4[None])
    out_a = _edge_final(g2a, e1a, eat_a, weights)
    out_b = _edge_final(g2b, e1b, eat_b, weights)
    return jnp.concatenate([out_a, out_b], axis=0)


# speedup vs baseline: 1.5202x; 1.5202x over previous
import functools

import jax
import jax.numpy as jnp
from jax import lax
from jax.experimental import pallas as pl
from jax.experimental.pallas import tpu as pltpu
from jax.experimental.pallas import tpu_sc as plsc

N = 10000
E = 320000
DN = 128
DE = 16
H = 80

NC, NS = 2, 16
NW = NC * NS
CH = 80
NCH_A = 77
NCH_B = 48
E_A = NW * NCH_A * CH
E_B = NW * NCH_B * CH
ROWS_A = E_A // CH
HP = 128
F32 = jnp.float32
BF = jnp.bfloat16

_SC_PARAMS = pltpu.CompilerParams(use_tc_tiling_on_sc=False)


@functools.cache
def _sc_mesh():
    return plsc.VectorSubcoreMesh(
        core_axis_name="c", subcore_axis_name="s", num_cores=NC, num_subcores=NS)


@functools.cache
def _gather_sum_kernel(nchunk, row0):
    per_w = nchunk * CH

    def body(a_hbm, b_hbm, dst_hbm, src_hbm, out_hbm,
             idxd, idxs, a0, b0, a1, b1, sem0, sem1):
        wid = lax.axis_index("s") * NC + lax.axis_index("c")
        r0 = row0 + wid * nchunk
        pltpu.sync_copy(dst_hbm.at[pl.ds(r0, nchunk)], idxd)
        pltpu.sync_copy(src_hbm.at[pl.ds(r0, nchunk)], idxs)
        base = wid * per_w

        def fire(j, ab, bb, sem):
            pltpu.async_copy(a_hbm.at[idxd.at[j]], ab, sem)
            pltpu.async_copy(b_hbm.at[idxs.at[j]], bb, sem)

        def drain(ab, bb, sem):
            pltpu.make_async_copy(a_hbm.at[idxd.at[0]], ab, sem).wait()
            pltpu.make_async_copy(b_hbm.at[idxs.at[0]], bb, sem).wait()

        def addstore(ab, bb, j):
            def row(i, c2):
                for q in range(HP // 16):
                    sl = pl.ds(q * 16, 16)
                    ab[i, sl] = ab[i, sl] + bb[i, sl]
                return c2

            lax.fori_loop(0, CH, row, 0)
            pltpu.sync_copy(ab, out_hbm.at[pl.ds(base + j * CH, CH)])

        fire(0, a0, b0, sem0)
        fire(1, a1, b1, sem1)

        def pair(i, carry):
            j0 = 2 * i
            drain(a0, b0, sem0)
            addstore(a0, b0, j0)

            @pl.when(j0 + 2 < nchunk)
            def _():
                fire(j0 + 2, a0, b0, sem0)

            drain(a1, b1, sem1)
            addstore(a1, b1, j0 + 1)

            @pl.when(j0 + 3 < nchunk)
            def _():
                fire(j0 + 3, a1, b1, sem1)

            return carry

        lax.fori_loop(0, nchunk // 2, pair, 0)
        if nchunk % 2:
            drain(a0, b0, sem0)
            addstore(a0, b0, nchunk - 1)

    return pl.kernel(
        body,
        out_type=jax.ShapeDtypeStruct((NW * per_w, HP), F32),
        mesh=_sc_mesh(),
        compiler_params=_SC_PARAMS,
        scratch_types=[
            pltpu.VMEM((nchunk, CH), jnp.int32),
            pltpu.VMEM((nchunk, CH), jnp.int32),
            pltpu.VMEM((CH, HP), F32),
            pltpu.VMEM((CH, HP), F32),
            pltpu.VMEM((CH, HP), F32),
            pltpu.VMEM((CH, HP), F32),
            pltpu.SemaphoreType.DMA,
            pltpu.SemaphoreType.DMA,
        ],
    )


@functools.cache
def _scatter_add_kernel(ncha, nchb):
    def body(ea_hbm, eb_hbm, dst_hbm, zero_hbm, out_hbm,
             idxa, idxb, e0, e1, shared, sem0, sem1):
        c = lax.axis_index("c")
        s = lax.axis_index("s")
        wid = s * NC + c

        @pl.when(s == 0)
        def _():
            pltpu.sync_copy(zero_hbm, shared)

        plsc.subcore_barrier()
        pltpu.sync_copy(dst_hbm.at[pl.ds(wid * ncha, ncha)], idxa)
        pltpu.sync_copy(dst_hbm.at[pl.ds(ROWS_A + wid * nchb, nchb)], idxb)

        def phase(e_hbm, idx, nch):
            base = wid * nch * CH

            def fire(j, eb, sem):
                pltpu.async_copy(e_hbm.at[pl.ds(base + j * CH, CH)], eb, sem)

            def drain(eb, sem):
                pltpu.make_async_copy(e_hbm.at[pl.ds(0, CH)], eb, sem).wait()

            def scat(eb, j):
                pltpu.sync_copy(eb, shared.at[idx.at[j]], add=True)

            fire(0, e0, sem0)
            fire(1, e1, sem1)

            def pair(i, carry):
                j0 = 2 * i
                drain(e0, sem0)
                scat(e0, j0)

                @pl.when(j0 + 2 < nch)
                def _():
                    fire(j0 + 2, e0, sem0)

                drain(e1, sem1)
                scat(e1, j0 + 1)

                @pl.when(j0 + 3 < nch)
                def _():
                    fire(j0 + 3, e1, sem1)

                return carry

            lax.fori_loop(0, nch // 2, pair, 0)
            if nch % 2:
                drain(e0, sem0)
                scat(e0, nch - 1)

        phase(ea_hbm, idxa, ncha)
        phase(eb_hbm, idxb, nchb)
        plsc.subcore_barrier()

        @pl.when(s == 0)
        def _():
            pltpu.sync_copy(shared, out_hbm.at[c])

    return pl.kernel(
        body,
        out_type=jax.ShapeDtypeStruct((NC, N, DE), F32),
        mesh=_sc_mesh(),
        compiler_params=_SC_PARAMS,
        scratch_types=[
            pltpu.VMEM((ncha, CH), jnp.int32),
            pltpu.VMEM((nchb, CH), jnp.int32),
            pltpu.VMEM((CH, DE), F32),
            pltpu.VMEM((CH, DE), F32),
            pltpu.VMEM_SHARED((N, DE), F32),
            pltpu.SemaphoreType.DMA,
            pltpu.SemaphoreType.DMA,
        ],
    )


def _dot(a, b):
    return jnp.dot(a, b, preferred_element_type=F32)


def _full(a):
    return pl.BlockSpec(a.shape, lambda i: (0,) * a.ndim)


def _project_body(x_ref, wd_ref, ws_ref, a_ref, b_ref):
    xb = x_ref[...]
    a_ref[...] = _dot(xb, wd_ref[...])
    b_ref[...] = _dot(xb, ws_ref[...])


def _node_project(x, wd, ws, blk=2000):
    return pl.pallas_call(
        _project_body,
        grid=(N // blk,),
        in_specs=[pl.BlockSpec((blk, DN), lambda i: (i, 0)),
                  _full(wd), _full(ws)],
        out_specs=[pl.BlockSpec((blk, HP), lambda i: (i, 0)),
                   pl.BlockSpec((blk, HP), lambda i: (i, 0))],
        out_shape=[jax.ShapeDtypeStruct((N, HP), F32),
                   jax.ShapeDtypeStruct((N, HP), F32)],
    )(x, wd, ws)


def _dot_tl(at, b):
    return lax.dot_general(at, b, (((0,), (0,)), ((), ())),
                           preferred_element_type=F32)


def _edge_body(g_ref, ea_ref, we_ref, b1_ref, w2_ref, b2_ref, w3_ref, b3_ref,
               w4_ref, b4_ref, out_ref):
    h = (g_ref[...] + _dot_tl(ea_ref[...].astype(BF), we_ref[...])
         + b1_ref[...])
    h = jnp.maximum(h, 0.0)
    h = jnp.maximum(_dot(h.astype(BF), w2_ref[...]) + b2_ref[...], 0.0)
    h = jnp.maximum(_dot(h.astype(BF), w3_ref[...]) + b3_ref[...], 0.0)
    out_ref[...] = _dot(h.astype(BF), w4_ref[...]) + b4_ref[...]


def _edge_mlp(g, ea_t, weights, blk=2560):
    ne = g.shape[0]
    return pl.pallas_call(
        _edge_body,
        grid=(ne // blk,),
        in_specs=[pl.BlockSpec((blk, HP), lambda i: (i, 0)),
                  pl.BlockSpec((DE, blk), lambda i: (0, i))]
                 + [_full(w) for w in weights],
        out_specs=pl.BlockSpec((blk, DE), lambda i: (i, 0)),
        out_shape=jax.ShapeDtypeStruct((ne, DE), F32),
    )(g, ea_t, *weights)


def _node_fused_body(x_ref, agg_ref, wx_ref, wa_ref, b1_ref, w2_ref, b2_ref,
                     w3_ref, b3_ref, w4_ref, b4_ref, wd_ref, ws_ref,
                     a_ref, b_ref):
    agg = agg_ref[0] + agg_ref[1]
    h = _dot(x_ref[...], wx_ref[...]) + _dot(agg, wa_ref[...]) + b1_ref[...]
    h = jnp.maximum(h, 0.0)
    h = jnp.maximum(_dot(h, w2_ref[...]) + b2_ref[...], 0.0)
    h = jnp.maximum(_dot(h, w3_ref[...]) + b3_ref[...], 0.0)
    x1 = _dot(h, w4_ref[...]) + b4_ref[...]
    a_ref[...] = _dot(x1, wd_ref[...])
    b_ref[...] = _dot(x1, ws_ref[...])


def _node_fused(x, agg2, weights, blk=2000):
    return pl.pallas_call(
        _node_fused_body,
        grid=(N // blk,),
        in_specs=[pl.BlockSpec((blk, DN), lambda i: (i, 0)),
                  pl.BlockSpec((NC, blk, DE), lambda i: (0, i, 0))]
                 + [_full(w) for w in weights],
        out_specs=[pl.BlockSpec((blk, HP), lambda i: (i, 0)),
                   pl.BlockSpec((blk, HP), lambda i: (i, 0))],
        out_shape=[jax.ShapeDtypeStruct((N, HP), F32),
                   jax.ShapeDtypeStruct((N, HP), F32)],
    )(x, agg2, *weights)


def _edge_final_body(g_ref, e1_ref, ea_ref, we_ref, b1_ref, w2_ref, b2_ref,
                     w3_ref, b3_ref, w4_ref, b4_ref, va_ref, vb_ref, vc_ref,
                     c1_ref, v2_ref, c2_ref, v3_ref, c3_ref, v4_ref, c4_ref,
                     out_ref):
    e1 = e1_ref[...].astype(BF)
    h = (g_ref[...] + _dot(e1, we_ref[...]) + b1_ref[...])
    h = jnp.maximum(h, 0.0)
    h = jnp.maximum(_dot(h.astype(BF), w2_ref[...]) + b2_ref[...], 0.0)
    h = jnp.maximum(_dot(h.astype(BF), w3_ref[...]) + b3_ref[...], 0.0)
    e2 = _dot(h.astype(BF), w4_ref[...]) + b4_ref[...]
    hw = (_dot_tl(ea_ref[...].astype(BF), va_ref[...]) + _dot(e1, vb_ref[...])
          + _dot(e2.astype(BF), vc_ref[...]) + c1_ref[...])
    hw = jnp.maximum(hw, 0.0)
    hw = jnp.maximum(_dot(hw.astype(BF), v2_ref[...]) + c2_ref[...], 0.0)
    hw = jnp.maximum(_dot(hw.astype(BF), v3_ref[...]) + c3_ref[...], 0.0)
    out_ref[...] = jax.nn.sigmoid(_dot(hw.astype(BF), v4_ref[...]) + c4_ref[...])


def _edge_final(g, e1, ea_t, weights, blk=2560):
    ne = g.shape[0]
    return pl.pallas_call(
        _edge_final_body,
        grid=(ne // blk,),
        in_specs=[pl.BlockSpec((blk, HP), lambda i: (i, 0)),
                  pl.BlockSpec((blk, DE), lambda i: (i, 0)),
                  pl.BlockSpec((DE, blk), lambda i: (0, i))]
                 + [_full(w) for w in weights],
        out_specs=pl.BlockSpec((blk, 1), lambda i: (i, 0)),
        out_shape=jax.ShapeDtypeStruct((ne, 1), F32),
    )(g, e1, ea_t, *weights)


def kernel(x, edge_index, edge_attr, params):
    dst2 = edge_index[1].reshape(E // CH, CH)
    src2 = edge_index[0].reshape(E // CH, CH)
    ea_t = edge_attr.T
    eat_a, eat_b = ea_t[:, :E_A], ea_t[:, E_A:]
    zeros = jnp.zeros((N, DE), F32)

    def padc(w):
        return jnp.concatenate(
            [w, jnp.zeros(w.shape[:-1] + (HP - w.shape[-1],), w.dtype)], -1)

    def padr(w):
        return jnp.concatenate(
            [w, jnp.zeros((HP - w.shape[0],) + w.shape[1:], w.dtype)], 0)

    (r1w1, r1b1), (r1w2, r1b2), (r1w3, r1b3), (r1w4, r1b4) = params['r1']
    (o1w1, o1b1), (o1w2, o1b2), (o1w3, o1b3), (o1w4, o1b4) = params['o1']
    (r2w1, r2b1), (r2w2, r2b2), (r2w3, r2b3), (r2w4, r2b4) = params['r2']
    (ww1, wb1), (ww2, wb2), (ww3, wb3), (ww4, wb4) = params['w']

    a1, b1 = _node_project(x, padc(r1w1[:DN]), padc(r1w1[DN:2 * DN]))
    g1a = _gather_sum_kernel(NCH_A, 0)(a1, b1, dst2, src2)
    g1b = _gather_sum_kernel(NCH_B, ROWS_A)(a1, b1, dst2, src2)
    r1ws = (padc(r1w1[2 * DN:]).astype(BF), padc(r1b1[None]),
            padr(r1w2).astype(BF), r1b2[None], r1w3.astype(BF), r1b3[None],
            r1w4.astype(BF), r1b4[None])
    e1a = _edge_mlp(g1a, eat_a, r1ws)
    e1b = _edge_mlp(g1b, eat_b, r1ws)
    agg2 = _scatter_add_kernel(NCH_A, NCH_B)(e1a, e1b, dst2, zeros)

    a2, b2 = _node_fused(x, agg2,
                         (o1w1[:DN], o1w1[DN:], o1b1[None], o1w2, o1b2[None],
                          o1w3, o1b3[None], o1w4, o1b4[None],
                          padc(r2w1[:DN]), padc(r2w1[DN:2 * DN])))

    g2a = _gather_sum_kernel(NCH_A, 0)(a2, b2, dst2, src2)
    g2b = _gather_sum_kernel(NCH_B, ROWS_A)(a2, b2, dst2, src2)
    weights = (padc(r2w1[2 * DN:]).astype(BF), padc(r2b1[None]),
               padr(r2w2).astype(BF), r2b2[None], r2w3.astype(BF), r2b3[None],
               r2w4.astype(BF), r2b4[None],
               ww1[:DE].astype(BF), ww1[DE:2 * DE].astype(BF),
               ww1[2 * DE:].astype(BF), wb1[None], ww2.astype(BF), wb2[None],
               ww3.astype(BF), wb3[None], ww4.astype(BF), wb4[None])
    out_a = _edge_final(g2a, e1a, eat_a, weights)
    out_b = _edge_final(g2b, e1b, eat_b, weights)
    return jnp.concatenate([out_a, out_b], axis=0)


# speedup vs baseline: 1.6363x; 1.0763x over previous
import functools

import jax
import jax.numpy as jnp
from jax import lax
from jax.experimental import pallas as pl
from jax.experimental.pallas import tpu as pltpu
from jax.experimental.pallas import tpu_sc as plsc

N = 10000
E = 320000
DN = 128
DE = 16
H = 80

NC, NS = 2, 16
NW = NC * NS
CH = 80
NCH_A = 77
NCH_B = 48
E_A = NW * NCH_A * CH
E_B = NW * NCH_B * CH
ROWS_A = E_A // CH
HP = 128
F32 = jnp.float32
BF = jnp.bfloat16

_SC_PARAMS = pltpu.CompilerParams(use_tc_tiling_on_sc=False)


@functools.cache
def _sc_mesh():
    return plsc.VectorSubcoreMesh(
        core_axis_name="c", subcore_axis_name="s", num_cores=NC, num_subcores=NS)


@functools.cache
def _gather_sum_kernel(nchunk, row0):
    per_w = nchunk * CH

    def body(a_hbm, b_hbm, dst_hbm, src_hbm, out_hbm,
             idxd, idxs, a0, b0, a1, b1, sem0, sem1):
        wid = lax.axis_index("s") * NC + lax.axis_index("c")
        r0 = row0 + wid * nchunk
        pltpu.sync_copy(dst_hbm.at[pl.ds(r0, nchunk)], idxd)
        pltpu.sync_copy(src_hbm.at[pl.ds(r0, nchunk)], idxs)
        base = wid * per_w

        def fire(j, ab, bb, sem):
            pltpu.async_copy(a_hbm.at[idxd.at[j]], ab, sem)
            pltpu.async_copy(b_hbm.at[idxs.at[j]], bb, sem)

        def drain(ab, bb, sem):
            pltpu.make_async_copy(a_hbm.at[idxd.at[0]], ab, sem).wait()
            pltpu.make_async_copy(b_hbm.at[idxs.at[0]], bb, sem).wait()

        def addstore(ab, bb, j):
            def row(i, c2):
                for q in range(HP // 16):
                    sl = pl.ds(q * 16, 16)
                    ab[i, sl] = ab[i, sl] + bb[i, sl]
                return c2

            lax.fori_loop(0, CH, row, 0)
            pltpu.sync_copy(ab, out_hbm.at[pl.ds(base + j * CH, CH)])

        fire(0, a0, b0, sem0)
        fire(1, a1, b1, sem1)

        def pair(i, carry):
            j0 = 2 * i
            drain(a0, b0, sem0)
            addstore(a0, b0, j0)

            @pl.when(j0 + 2 < nchunk)
            def _():
                fire(j0 + 2, a0, b0, sem0)

            drain(a1, b1, sem1)
            addstore(a1, b1, j0 + 1)

            @pl.when(j0 + 3 < nchunk)
            def _():
                fire(j0 + 3, a1, b1, sem1)

            return carry

        lax.fori_loop(0, nchunk // 2, pair, 0)
        if nchunk % 2:
            drain(a0, b0, sem0)
            addstore(a0, b0, nchunk - 1)

    return pl.kernel(
        body,
        out_type=jax.ShapeDtypeStruct((NW * per_w, HP), F32),
        mesh=_sc_mesh(),
        compiler_params=_SC_PARAMS,
        scratch_types=[
            pltpu.VMEM((nchunk, CH), jnp.int32),
            pltpu.VMEM((nchunk, CH), jnp.int32),
            pltpu.VMEM((CH, HP), F32),
            pltpu.VMEM((CH, HP), F32),
            pltpu.VMEM((CH, HP), F32),
            pltpu.VMEM((CH, HP), F32),
            pltpu.SemaphoreType.DMA,
            pltpu.SemaphoreType.DMA,
        ],
    )


@functools.cache
def _scatter_add_kernel(ncha, nchb):
    def body(ea_hbm, eb_hbm, dst_hbm, zero_hbm, out_hbm,
             idxa, idxb, e0, e1, shared, sem0, sem1):
        c = lax.axis_index("c")
        s = lax.axis_index("s")
        wid = s * NC + c

        @pl.when(s == 0)
        def _():
            pltpu.sync_copy(zero_hbm, shared)

        plsc.subcore_barrier()
        pltpu.sync_copy(dst_hbm.at[pl.ds(wid * ncha, ncha)], idxa)
        pltpu.sync_copy(dst_hbm.at[pl.ds(ROWS_A + wid * nchb, nchb)], idxb)

        def phase(e_hbm, idx, nch):
            base = wid * nch * CH

            def fire(j, eb, sem):
                pltpu.async_copy(
                    e_hbm.at[pl.ds(base + j * CH, CH), pl.ds(0, DE)], eb, sem)

            def drain(eb, sem):
                pltpu.make_async_copy(
                    e_hbm.at[pl.ds(0, CH), pl.ds(0, DE)], eb, sem).wait()

            def scat(eb, j):
                pltpu.sync_copy(eb, shared.at[idx.at[j]], add=True)

            fire(0, e0, sem0)
            fire(1, e1, sem1)

            def pair(i, carry):
                j0 = 2 * i
                drain(e0, sem0)
                scat(e0, j0)

                @pl.when(j0 + 2 < nch)
                def _():
                    fire(j0 + 2, e0, sem0)

                drain(e1, sem1)
                scat(e1, j0 + 1)

                @pl.when(j0 + 3 < nch)
                def _():
                    fire(j0 + 3, e1, sem1)

                return carry

            lax.fori_loop(0, nch // 2, pair, 0)
            if nch % 2:
                drain(e0, sem0)
                scat(e0, nch - 1)

        phase(ea_hbm, idxa, ncha)
        phase(eb_hbm, idxb, nchb)
        plsc.subcore_barrier()

        @pl.when(s == 0)
        def _():
            pltpu.sync_copy(shared, out_hbm.at[c])

    return pl.kernel(
        body,
        out_type=jax.ShapeDtypeStruct((NC, N, DE), F32),
        mesh=_sc_mesh(),
        compiler_params=_SC_PARAMS,
        scratch_types=[
            pltpu.VMEM((ncha, CH), jnp.int32),
            pltpu.VMEM((nchb, CH), jnp.int32),
            pltpu.VMEM((CH, DE), F32),
            pltpu.VMEM((CH, DE), F32),
            pltpu.VMEM_SHARED((N, DE), F32),
            pltpu.SemaphoreType.DMA,
            pltpu.SemaphoreType.DMA,
        ],
    )


def _dot(a, b):
    return jnp.dot(a, b, preferred_element_type=F32)


def _full(a):
    return pl.BlockSpec(a.shape, lambda i: (0,) * a.ndim)


def _project_body(x_ref, wd_ref, ws_ref, a_ref, b_ref):
    xb = x_ref[...]
    a_ref[...] = _dot(xb, wd_ref[...])
    b_ref[...] = _dot(xb, ws_ref[...])


def _node_project(x, wd, ws, blk=2000):
    return pl.pallas_call(
        _project_body,
        grid=(N // blk,),
        in_specs=[pl.BlockSpec((blk, DN), lambda i: (i, 0)),
                  _full(wd), _full(ws)],
        out_specs=[pl.BlockSpec((blk, HP), lambda i: (i, 0)),
                   pl.BlockSpec((blk, HP), lambda i: (i, 0))],
        out_shape=[jax.ShapeDtypeStruct((N, HP), F32),
                   jax.ShapeDtypeStruct((N, HP), F32)],
    )(x, wd, ws)


def _dot_tl(at, b):
    return lax.dot_general(at, b, (((0,), (0,)), ((), ())),
                           preferred_element_type=F32)


def _edge_body(g_ref, ea_ref, we_ref, b1_ref, w2_ref, b2_ref, w3_ref, b3_ref,
               w4_ref, b4_ref, out_ref):
    h = g_ref[...] + _dot_tl(ea_ref[...], we_ref[...]) + b1_ref[...]
    h = jnp.maximum(h, 0.0)
    h = jnp.maximum(_dot(h, w2_ref[...]) + b2_ref[...], 0.0)
    h = jnp.maximum(_dot(h, w3_ref[...]) + b3_ref[...], 0.0)
    out_ref[...] = _dot(h, w4_ref[...]) + b4_ref[...]


def _edge_mlp(g, ea_t, weights, blk=2560):
    ne = g.shape[0]
    return pl.pallas_call(
        _edge_body,
        grid=(ne // blk,),
        in_specs=[pl.BlockSpec((blk, HP), lambda i: (i, 0)),
                  pl.BlockSpec((DE, blk), lambda i: (0, i))]
                 + [_full(w) for w in weights],
        out_specs=pl.BlockSpec((blk, HP), lambda i: (i, 0)),
        out_shape=jax.ShapeDtypeStruct((ne, HP), F32),
    )(g, ea_t, *weights)


def _node_fused_body(x_ref, agg_ref, wx_ref, wa_ref, b1_ref, w2_ref, b2_ref,
                     w3_ref, b3_ref, w4_ref, b4_ref, wd_ref, ws_ref,
                     a_ref, b_ref):
    agg = agg_ref[0] + agg_ref[1]
    h = _dot(x_ref[...], wx_ref[...]) + _dot(agg, wa_ref[...]) + b1_ref[...]
    h = jnp.maximum(h, 0.0)
    h = jnp.maximum(_dot(h, w2_ref[...]) + b2_ref[...], 0.0)
    h = jnp.maximum(_dot(h, w3_ref[...]) + b3_ref[...], 0.0)
    x1 = _dot(h, w4_ref[...]) + b4_ref[...]
    a_ref[...] = _dot(x1, wd_ref[...])
    b_ref[...] = _dot(x1, ws_ref[...])


def _node_fused(x, agg2, weights, blk=2000):
    return pl.pallas_call(
        _node_fused_body,
        grid=(N // blk,),
        in_specs=[pl.BlockSpec((blk, DN), lambda i: (i, 0)),
                  pl.BlockSpec((NC, blk, DE), lambda i: (0, i, 0))]
                 + [_full(w) for w in weights],
        out_specs=[pl.BlockSpec((blk, HP), lambda i: (i, 0)),
                   pl.BlockSpec((blk, HP), lambda i: (i, 0))],
        out_shape=[jax.ShapeDtypeStruct((N, HP), F32),
                   jax.ShapeDtypeStruct((N, HP), F32)],
    )(x, agg2, *weights)


def _edge_final_body(g_ref, e1_ref, ea_ref, we_ref, b1_ref, w2_ref, b2_ref,
                     w3_ref, b3_ref, w4_ref, b4_ref, va_ref, vb_ref, vc_ref,
                     c1_ref, v2_ref, c2_ref, v3_ref, c3_ref, v4_ref, c4_ref,
                     out_ref):
    e1 = e1_ref[...]
    h = g_ref[...] + _dot(e1, we_ref[...]) + b1_ref[...]
    h = jnp.maximum(h, 0.0)
    h = jnp.maximum(_dot(h, w2_ref[...]) + b2_ref[...], 0.0)
    h = jnp.maximum(_dot(h, w3_ref[...]) + b3_ref[...], 0.0)
    e2 = _dot(h, w4_ref[...]) + b4_ref[...]
    hw = (_dot_tl(ea_ref[...], va_ref[...]) + _dot(e1, vb_ref[...])
          + _dot(e2, vc_ref[...]) + c1_ref[...])
    hw = jnp.maximum(hw, 0.0)
    hw = jnp.maximum(_dot(hw, v2_ref[...]) + c2_ref[...], 0.0)
    hw = jnp.maximum(_dot(hw, v3_ref[...]) + c3_ref[...], 0.0)
    out_ref[...] = jax.nn.sigmoid(_dot(hw, v4_ref[...]) + c4_ref[...])


def _edge_final(g, e1, ea_t, weights, blk=2560):
    ne = g.shape[0]
    return pl.pallas_call(
        _edge_final_body,
        grid=(ne // blk,),
        in_specs=[pl.BlockSpec((blk, HP), lambda i: (i, 0)),
                  pl.BlockSpec((blk, HP), lambda i: (i, 0)),
                  pl.BlockSpec((DE, blk), lambda i: (0, i))]
                 + [_full(w) for w in weights],
        out_specs=pl.BlockSpec((blk, 1), lambda i: (i, 0)),
        out_shape=jax.ShapeDtypeStruct((ne, 1), F32),
    )(g, e1, ea_t, *weights)


def kernel(x, edge_index, edge_attr, params):
    dst2 = edge_index[1].reshape(E // CH, CH)
    src2 = edge_index[0].reshape(E // CH, CH)
    ea_t = edge_attr.T
    eat_a, eat_b = ea_t[:, :E_A], ea_t[:, E_A:]
    zeros = jnp.zeros((N, DE), F32)

    def padc(w):
        return jnp.concatenate(
            [w, jnp.zeros(w.shape[:-1] + (HP - w.shape[-1],), w.dtype)], -1)

    def padr(w):
        return jnp.concatenate(
            [w, jnp.zeros((HP - w.shape[0],) + w.shape[1:], w.dtype)], 0)

    (r1w1, r1b1), (r1w2, r1b2), (r1w3, r1b3), (r1w4, r1b4) = params['r1']
    (o1w1, o1b1), (o1w2, o1b2), (o1w3, o1b3), (o1w4, o1b4) = params['o1']
    (r2w1, r2b1), (r2w2, r2b2), (r2w3, r2b3), (r2w4, r2b4) = params['r2']
    (ww1, wb1), (ww2, wb2), (ww3, wb3), (ww4, wb4) = params['w']

    a1, b1 = _node_project(x, padc(r1w1[:DN]), padc(r1w1[DN:2 * DN]))
    g1a = _gather_sum_kernel(NCH_A, 0)(a1, b1, dst2, src2)
    g1b = _gather_sum_kernel(NCH_B, ROWS_A)(a1, b1, dst2, src2)
    r1ws = (padc(r1w1[2 * DN:]), padc(r1b1[None]), padr(r1w2), r1b2[None],
            r1w3, r1b3[None], padc(r1w4), padc(r1b4[None]))
    e1a = _edge_mlp(g1a, eat_a, r1ws)
    e1b = _edge_mlp(g1b, eat_b, r1ws)
    agg2 = _scatter_add_kernel(NCH_A, NCH_B)(e1a, e1b, dst2, zeros)

    a2, b2 = _node_fused(x, agg2,
                         (o1w1[:DN], o1w1[DN:], o1b1[None], o1w2, o1b2[None],
                          o1w3, o1b3[None], o1w4, o1b4[None],
                          padc(r2w1[:DN]), padc(r2w1[DN:2 * DN])))

    g2a = _gather_sum_kernel(NCH_A, 0)(a2, b2, dst2, src2)
    g2b = _gather_sum_kernel(NCH_B, ROWS_A)(a2, b2, dst2, src2)
    weights = (padr(padc(r2w1[2 * DN:])), padc(r2b1[None]), padr(r2w2),
               r2b2[None], r2w3, r2b3[None], r2w4, r2b4[None],
               ww1[:DE], padr(ww1[DE:2 * DE]), ww1[2 * DE:],
               wb1[None], ww2, wb2[None], ww3, wb3[None], ww4, wb4[None])
    out_a = _edge_final(g2a, e1a, eat_a, weights)
    out_b = _edge_final(g2b, e1b, eat_b, weights)
    return jnp.concatenate([out_a, out_b], axis=0)


# speedup vs baseline: 1.6379x; 1.0010x over previous
import functools

import jax
import jax.numpy as jnp
from jax import lax
from jax.experimental import pallas as pl
from jax.experimental.pallas import tpu as pltpu
from jax.experimental.pallas import tpu_sc as plsc

N = 10000
E = 320000
DN = 128
DE = 16
H = 80

NC, NS = 2, 16
NW = NC * NS
CH = 80
NCH_A = 77
NCH_B = 48
E_A = NW * NCH_A * CH
E_B = NW * NCH_B * CH
ROWS_A = E_A // CH
HP = 128
F32 = jnp.float32

_SC_PARAMS = pltpu.CompilerParams(use_tc_tiling_on_sc=False)


@functools.cache
def _sc_mesh():
    return plsc.VectorSubcoreMesh(
        core_axis_name="c", subcore_axis_name="s", num_cores=NC, num_subcores=NS)


@functools.cache
def _gather_sum_kernel(nchunk, row0):
    per_w = nchunk * CH

    def body(a_hbm, b_hbm, dst_hbm, src_hbm, out_hbm,
             idxd, idxs, a0, b0, a1, b1, sem0, sem1):
        wid = lax.axis_index("s") * NC + lax.axis_index("c")
        r0 = row0 + wid * nchunk
        pltpu.sync_copy(dst_hbm.at[pl.ds(r0, nchunk)], idxd)
        pltpu.sync_copy(src_hbm.at[pl.ds(r0, nchunk)], idxs)
        base = wid * per_w

        def fire(j, ab, bb, sem):
            pltpu.async_copy(a_hbm.at[idxd.at[j]], ab, sem)
            pltpu.async_copy(b_hbm.at[idxs.at[j]], bb, sem)

        def drain(ab, bb, sem):
            pltpu.make_async_copy(a_hbm.at[idxd.at[0]], ab, sem).wait()
            pltpu.make_async_copy(b_hbm.at[idxs.at[0]], bb, sem).wait()

        def addstore(ab, bb, j):
            def row(i, c2):
                for q in range(HP // 16):
                    sl = pl.ds(q * 16, 16)
                    ab[i, sl] = ab[i, sl] + bb[i, sl]
                return c2

            lax.fori_loop(0, CH, row, 0)
            pltpu.sync_copy(ab, out_hbm.at[pl.ds(base + j * CH, CH)])

        fire(0, a0, b0, sem0)
        fire(1, a1, b1, sem1)

        def pair(i, carry):
            j0 = 2 * i
            drain(a0, b0, sem0)
            addstore(a0, b0, j0)

            @pl.when(j0 + 2 < nchunk)
            def _():
                fire(j0 + 2, a0, b0, sem0)

            drain(a1, b1, sem1)
            addstore(a1, b1, j0 + 1)

            @pl.when(j0 + 3 < nchunk)
            def _():
                fire(j0 + 3, a1, b1, sem1)

            return carry

        lax.fori_loop(0, nchunk // 2, pair, 0)
        if nchunk % 2:
            drain(a0, b0, sem0)
            addstore(a0, b0, nchunk - 1)

    return pl.kernel(
        body,
        out_type=jax.ShapeDtypeStruct((NW * per_w, HP), F32),
        mesh=_sc_mesh(),
        compiler_params=_SC_PARAMS,
        scratch_types=[
            pltpu.VMEM((nchunk, CH), jnp.int32),
            pltpu.VMEM((nchunk, CH), jnp.int32),
            pltpu.VMEM((CH, HP), F32),
            pltpu.VMEM((CH, HP), F32),
            pltpu.VMEM((CH, HP), F32),
            pltpu.VMEM((CH, HP), F32),
            pltpu.SemaphoreType.DMA,
            pltpu.SemaphoreType.DMA,
        ],
    )


@functools.cache
def _scatter_add_kernel(ncha, nchb):
    def body(ea_hbm, eb_hbm, dst_hbm, zero_hbm, out_hbm,
             idxa, idxb, e0, e1, shared, sem0, sem1):
        c = lax.axis_index("c")
        s = lax.axis_index("s")
        wid = s * NC + c

        @pl.when(s == 0)
        def _():
            pltpu.sync_copy(zero_hbm, shared)

        plsc.subcore_barrier()
        pltpu.sync_copy(dst_hbm.at[pl.ds(wid * ncha, ncha)], idxa)
        pltpu.sync_copy(dst_hbm.at[pl.ds(ROWS_A + wid * nchb, nchb)], idxb)

        def phase(e_hbm, idx, nch):
            base = wid * nch * CH

            def fire(j, eb, sem):
                pltpu.async_copy(
                    e_hbm.at[pl.ds(base + j * CH, CH), pl.ds(0, DE)], eb, sem)

            def drain(eb, sem):
                pltpu.make_async_copy(
                    e_hbm.at[pl.ds(0, CH), pl.ds(0, DE)], eb, sem).wait()

            def scat(eb, j):
                pltpu.sync_copy(eb, shared.at[idx.at[j]], add=True)

            fire(0, e0, sem0)
            fire(1, e1, sem1)

            def pair(i, carry):
                j0 = 2 * i
                drain(e0, sem0)
                scat(e0, j0)

                @pl.when(j0 + 2 < nch)
                def _():
                    fire(j0 + 2, e0, sem0)

                drain(e1, sem1)
                scat(e1, j0 + 1)

                @pl.when(j0 + 3 < nch)
                def _():
                    fire(j0 + 3, e1, sem1)

                return carry

            lax.fori_loop(0, nch // 2, pair, 0)
            if nch % 2:
                drain(e0, sem0)
                scat(e0, nch - 1)

        phase(ea_hbm, idxa, ncha)
        phase(eb_hbm, idxb, nchb)
        plsc.subcore_barrier()

        @pl.when(s == 0)
        def _():
            pltpu.sync_copy(shared, out_hbm.at[c])

    return pl.kernel(
        body,
        out_type=jax.ShapeDtypeStruct((NC, N, DE), F32),
        mesh=_sc_mesh(),
        compiler_params=_SC_PARAMS,
        scratch_types=[
            pltpu.VMEM((ncha, CH), jnp.int32),
            pltpu.VMEM((nchb, CH), jnp.int32),
            pltpu.VMEM((CH, DE), F32),
            pltpu.VMEM((CH, DE), F32),
            pltpu.VMEM_SHARED((N, DE), F32),
            pltpu.SemaphoreType.DMA,
            pltpu.SemaphoreType.DMA,
        ],
    )


def _dot(a, b):
    return jnp.dot(a, b, preferred_element_type=F32)


def _full(a):
    return pl.BlockSpec(a.shape, lambda i: (0,) * a.ndim)


def _project_body(x_ref, wd_ref, ws_ref, a_ref, b_ref):
    xb = x_ref[...]
    a_ref[...] = _dot(xb, wd_ref[...])
    b_ref[...] = _dot(xb, ws_ref[...])


def _node_project(x, wd, ws, blk=2000):
    return pl.pallas_call(
        _project_body,
        grid=(N // blk,),
        in_specs=[pl.BlockSpec((blk, DN), lambda i: (i, 0)),
                  _full(wd), _full(ws)],
        out_specs=[pl.BlockSpec((blk, HP), lambda i: (i, 0)),
                   pl.BlockSpec((blk, HP), lambda i: (i, 0))],
        out_shape=[jax.ShapeDtypeStruct((N, HP), F32),
                   jax.ShapeDtypeStruct((N, HP), F32)],
    )(x, wd, ws)


def _dot_tl(at, b):
    return lax.dot_general(at, b, (((0,), (0,)), ((), ())),
                           preferred_element_type=F32)


def _edge_body(g_ref, ea_ref, we_ref, b1_ref, w2_ref, b2_ref, w3_ref, b3_ref,
               w4_ref, b4_ref, out_ref):
    h = g_ref[...] + _dot_tl(ea_ref[...], we_ref[...]) + b1_ref[...]
    h = jnp.maximum(h, 0.0)
    h = jnp.maximum(_dot(h, w2_ref[...]) + b2_ref[...], 0.0)
    h = jnp.maximum(_dot(h, w3_ref[...]) + b3_ref[...], 0.0)
    out_ref[...] = _dot(h, w4_ref[...]) + b4_ref[...]


def _edge_mlp(g, ea_t, weights, blk=2560):
    ne = g.shape[0]
    return pl.pallas_call(
        _edge_body,
        grid=(ne // blk,),
        in_specs=[pl.BlockSpec((blk, HP), lambda i: (i, 0)),
                  pl.BlockSpec((DE, blk), lambda i: (0, i))]
                 + [_full(w) for w in weights],
        out_specs=pl.BlockSpec((blk, HP), lambda i: (i, 0)),
        out_shape=jax.ShapeDtypeStruct((ne, HP), F32),
    )(g, ea_t, *weights)


def _node_fused_body(x_ref, agg_ref, wx_ref, wa_ref, b1_ref, w2_ref, b2_ref,
                     w3_ref, b3_ref, w4_ref, b4_ref, wd_ref, ws_ref,
                     a_ref, b_ref):
    agg = agg_ref[0] + agg_ref[1]
    h = _dot(x_ref[...], wx_ref[...]) + _dot(agg, wa_ref[...]) + b1_ref[...]
    h = jnp.maximum(h, 0.0)
    h = jnp.maximum(_dot(h, w2_ref[...]) + b2_ref[...], 0.0)
    h = jnp.maximum(_dot(h, w3_ref[...]) + b3_ref[...], 0.0)
    x1 = _dot(h, w4_ref[...]) + b4_ref[...]
    a_ref[...] = _dot(x1, wd_ref[...])
    b_ref[...] = _dot(x1, ws_ref[...])


def _node_fused(x, agg2, weights, blk=2000):
    return pl.pallas_call(
        _node_fused_body,
        grid=(N // blk,),
        in_specs=[pl.BlockSpec((blk, DN), lambda i: (i, 0)),
                  pl.BlockSpec((NC, blk, DE), lambda i: (0, i, 0))]
                 + [_full(w) for w in weights],
        out_specs=[pl.BlockSpec((blk, HP), lambda i: (i, 0)),
                   pl.BlockSpec((blk, HP), lambda i: (i, 0))],
        out_shape=[jax.ShapeDtypeStruct((N, HP), F32),
                   jax.ShapeDtypeStruct((N, HP), F32)],
    )(x, agg2, *weights)


def _edge_final_body(g_ref, e1_ref, ea_ref, we_ref, b1_ref, w2_ref, b2_ref,
                     w3_ref, b3_ref, w4_ref, b4_ref, va_ref, vb_ref, vc_ref,
                     c1_ref, v2_ref, c2_ref, v3_ref, c3_ref, v4_ref, c4_ref,
                     out_ref):
    e1 = e1_ref[...]
    h = g_ref[...] + _dot(e1, we_ref[...]) + b1_ref[...]
    h = jnp.maximum(h, 0.0)
    h = jnp.maximum(_dot(h, w2_ref[...]) + b2_ref[...], 0.0)
    h = jnp.maximum(_dot(h, w3_ref[...]) + b3_ref[...], 0.0)
    e2 = _dot(h, w4_ref[...]) + b4_ref[...]
    hw = (_dot_tl(ea_ref[...], va_ref[...]) + _dot(e1, vb_ref[...])
          + _dot(e2, vc_ref[...]) + c1_ref[...])
    hw = jnp.maximum(hw, 0.0)
    hw = jnp.maximum(_dot(hw, v2_ref[...]) + c2_ref[...], 0.0)
    hw = jnp.maximum(_dot(hw, v3_ref[...]) + c3_ref[...], 0.0)
    out_ref[...] = jax.nn.sigmoid(_dot(hw, v4_ref[...]) + c4_ref[...])


def _edge_final(g, e1, ea_t, weights, blk=2560):
    ne = g.shape[0]
    return pl.pallas_call(
        _edge_final_body,
        grid=(ne // blk,),
        in_specs=[pl.BlockSpec((blk, HP), lambda i: (i, 0)),
                  pl.BlockSpec((blk, HP), lambda i: (i, 0)),
                  pl.BlockSpec((DE, blk), lambda i: (0, i))]
                 + [_full(w) for w in weights],
        out_specs=pl.BlockSpec((blk, 1), lambda i: (i, 0)),
        out_shape=jax.ShapeDtypeStruct((ne, 1), F32),
    )(g, e1, ea_t, *weights)


def kernel(x, edge_index, edge_attr, params):
    dst2 = edge_index[1].reshape(E // CH, CH)
    src2 = edge_index[0].reshape(E // CH, CH)
    ea_t = edge_attr.T
    eat_a, eat_b = ea_t[:, :E_A], ea_t[:, E_A:]
    zeros = jnp.zeros((N, DE), F32)

    def padc(w):
        return jnp.concatenate(
            [w, jnp.zeros(w.shape[:-1] + (HP - w.shape[-1],), w.dtype)], -1)

    def padr(w):
        return jnp.concatenate(
            [w, jnp.zeros((HP - w.shape[0],) + w.shape[1:], w.dtype)], 0)

    (r1w1, r1b1), (r1w2, r1b2), (r1w3, r1b3), (r1w4, r1b4) = params['r1']
    (o1w1, o1b1), (o1w2, o1b2), (o1w3, o1b3), (o1w4, o1b4) = params['o1']
    (r2w1, r2b1), (r2w2, r2b2), (r2w3, r2b3), (r2w4, r2b4) = params['r2']
    (ww1, wb1), (ww2, wb2), (ww3, wb3), (ww4, wb4) = params['w']

    a1, b1 = _node_project(x, padc(r1w1[:DN]), padc(r1w1[DN:2 * DN]))
    g1a = _gather_sum_kernel(NCH_A, 0)(a1, b1, dst2, src2)
    g1b = _gather_sum_kernel(NCH_B, ROWS_A)(a1, b1, dst2, src2)
    r1ws = (padc(r1w1[2 * DN:]), padc(r1b1[None]), padr(r1w2), r1b2[None],
            r1w3, r1b3[None], padc(r1w4), padc(r1b4[None]))
    e1a = _edge_mlp(g1a, eat_a, r1ws)
    e1b = _edge_mlp(g1b, eat_b, r1ws)
    agg2 = _scatter_add_kernel(NCH_A, NCH_B)(e1a, e1b, dst2, zeros)

    a2, b2 = _node_fused(x, agg2,
                         (o1w1[:DN], o1w1[DN:], o1b1[None], o1w2, o1b2[None],
                          o1w3, o1b3[None], o1w4, o1b4[None],
                          padc(r2w1[:DN]), padc(r2w1[DN:2 * DN])))

    g2a = _gather_sum_kernel(NCH_A, 0)(a2, b2, dst2, src2)
    g2b = _gather_sum_kernel(NCH_B, ROWS_A)(a2, b2, dst2, src2)
    weights = (padr(padc(r2w1[2 * DN:])), padc(r2b1[None]), padr(r2w2),
               r2b2[None], r2w3, r2b3[None], r2w4, r2b4[None],
               ww1[:DE], padr(ww1[DE:2 * DE]), ww1[2 * DE:],
               wb1[None], ww2, wb2[None], ww3, wb3[None], ww4, wb4[None])
    out_a = _edge_final(g2a, e1a, eat_a, weights)
    out_b = _edge_final(g2b, e1b, eat_b, weights)
    return jnp.concatenate([out_a, out_b], axis=0)
